# Initial kernel scaffold; baseline (speedup 1.0000x reference)
import jax
import jax.numpy as jnp
from jax.experimental import pallas as pl


def kernel(node_wids, edge_index, root_idxs, emb_table, W_self, W_msg, b):
    raise NotImplementedError("write your pallas kernel here")


# speedup vs baseline: 2.9136x; 2.9136x over previous
import functools

import jax
import jax.numpy as jnp
from jax import lax
from jax.experimental import pallas as pl
from jax.experimental.pallas import tpu as pltpu
from jax.experimental.pallas import tpu_sc as plsc

N_NODES = 10000
HIDDEN = 128
NUM_LAYERS = 3
BATCH = 256

_NC, _NS = 2, 16
_NW = _NC * _NS
_EC = 128
_CHUNKS = 80
_EPAD = _NW * _CHUNKS * _EC
_NPAD = 10240
_RPT = _NPAD // _NS
_GC = 3
_GPAD = _NW * _GC * _EC
_BPW = BATCH // _NW

_MESH = plsc.VectorSubcoreMesh(core_axis_name="c", subcore_axis_name="s")


def _f32(*shape):
    return jax.ShapeDtypeStruct(shape, jnp.float32)


def _sc_emb_gather(table, idx2d):
    @functools.partial(
        pl.kernel,
        out_type=_f32(_GPAD, HIDDEN),
        mesh=_MESH,
        scratch_types=[
            pltpu.VMEM((_GC, _EC), jnp.int32),
            pltpu.VMEM((_EC, HIDDEN), jnp.float32),
            pltpu.SemaphoreType.DMA,
        ],
    )
    def k(tab_hbm, idx_hbm, out_hbm, idx_v, rows_v, sem):
        w = lax.axis_index("s") * _NC + lax.axis_index("c")
        pltpu.sync_copy(idx_hbm.at[w], idx_v)

        @pl.loop(0, _GC)
        def _(j):
            pltpu.async_copy(tab_hbm.at[idx_v.at[j]], rows_v, sem).wait()
            pltpu.sync_copy(rows_v, out_hbm.at[pl.ds((w * _GC + j) * _EC, _EC)])

    return k(table, idx2d)


def _sc_edge_agg(hm, src2d, dst2d, zstripe):
    @functools.partial(
        pl.kernel,
        out_type=_f32(2 * _NPAD, HIDDEN),
        mesh=_MESH,
        scratch_types=[
            pltpu.VMEM((_CHUNKS, _EC), jnp.int32),
            pltpu.VMEM((_CHUNKS, _EC), jnp.int32),
            pltpu.VMEM((_EC, HIDDEN), jnp.float32),
            pltpu.VMEM_SHARED((_NPAD, HIDDEN), jnp.float32),
            pltpu.SemaphoreType.DMA,
        ],
    )
    def k(hm_hbm, src_hbm, dst_hbm, z_hbm, out_hbm, src_v, dst_v, rows_v,
          agg_sh, sem):
        c = lax.axis_index("c")
        s = lax.axis_index("s")
        w = s * _NC + c
        pltpu.sync_copy(z_hbm, agg_sh.at[pl.ds(s * _RPT, _RPT)])
        pltpu.sync_copy(src_hbm.at[pl.ds(w * _CHUNKS, _CHUNKS)], src_v)
        pltpu.sync_copy(dst_hbm.at[pl.ds(w * _CHUNKS, _CHUNKS)], dst_v)
        plsc.subcore_barrier()

        @pl.loop(0, _CHUNKS)
        def _(j):
            pltpu.async_copy(hm_hbm.at[src_v.at[j]], rows_v, sem).wait()
            pltpu.sync_copy(rows_v, agg_sh.at[dst_v.at[j]], add=True)

        plsc.subcore_barrier()
        pltpu.sync_copy(agg_sh.at[pl.ds(s * _RPT, _RPT)],
                        out_hbm.at[pl.ds(c * _NPAD + s * _RPT, _RPT)])

    return k(hm, src2d, dst2d, zstripe)


def _sc_root_gather(h, roots):
    @functools.partial(
        pl.kernel,
        out_type=_f32(BATCH, HIDDEN),
        mesh=_MESH,
        scratch_types=[
            pltpu.VMEM((_BPW,), jnp.int32),
            pltpu.VMEM((_BPW, HIDDEN), jnp.float32),
            pltpu.SemaphoreType.DMA,
        ],
    )
    def k(h_hbm, r_hbm, out_hbm, idx_v, rows_v, sem):
        w = lax.axis_index("s") * _NC + lax.axis_index("c")
        pltpu.sync_copy(r_hbm.at[pl.ds(w * _BPW, _BPW)], idx_v)
        pltpu.async_copy(h_hbm.at[idx_v], rows_v, sem).wait()
        pltpu.sync_copy(rows_v, out_hbm.at[pl.ds(w * _BPW, _BPW)])

    return k(h, roots)


_BLK = 1000
_NBLK = N_NODES // _BLK

_row_spec = pl.BlockSpec((_BLK, HIDDEN), lambda i: (i, 0))
_w_spec = pl.BlockSpec((HIDDEN, HIDDEN), lambda i: (0, 0))
_b_spec = pl.BlockSpec((1, HIDDEN), lambda i: (0, 0))


def _tc_mm_first(h0, Ws, Wm, bias):
    def body(h_ref, ws_ref, wm_ref, b_ref, hs_ref, hm_ref):
        hb = h_ref[...]
        hs_ref[...] = (jnp.dot(hb, ws_ref[...],
                               preferred_element_type=jnp.float32)
                       + b_ref[...])
        hm_ref[...] = jnp.dot(hb, wm_ref[...],
                              preferred_element_type=jnp.float32)

    return pl.pallas_call(
        body,
        grid=(_NBLK,),
        in_specs=[_row_spec, _w_spec, _w_spec, _b_spec],
        out_specs=[_row_spec, _row_spec],
        out_shape=[_f32(N_NODES, HIDDEN)] * 2,
    )(h0, Ws, Wm, bias)


def _tc_mm_mid(hs_in, a0, a1, Ws, Wm, bias):
    def body(hs_ref, a0_ref, a1_ref, ws_ref, wm_ref, b_ref, hs_o, hm_o):
        h = jnp.maximum(hs_ref[...] + a0_ref[...] + a1_ref[...], 0.0)
        hs_o[...] = (jnp.dot(h, ws_ref[...],
                             preferred_element_type=jnp.float32)
                     + b_ref[...])
        hm_o[...] = jnp.dot(h, wm_ref[...],
                            preferred_element_type=jnp.float32)

    return pl.pallas_call(
        body,
        grid=(_NBLK,),
        in_specs=[_row_spec, _row_spec, _row_spec, _w_spec, _w_spec, _b_spec],
        out_specs=[_row_spec, _row_spec],
        out_shape=[_f32(N_NODES, HIDDEN)] * 2,
    )(hs_in, a0, a1, Ws, Wm, bias)


def _tc_relu_sum(hs_in, a0, a1):
    def body(hs_ref, a0_ref, a1_ref, h_o):
        h_o[...] = jnp.maximum(hs_ref[...] + a0_ref[...] + a1_ref[...], 0.0)

    return pl.pallas_call(
        body,
        grid=(_NBLK,),
        in_specs=[_row_spec, _row_spec, _row_spec],
        out_specs=_row_spec,
        out_shape=_f32(N_NODES, HIDDEN),
    )(hs_in, a0, a1)


def kernel(node_wids, edge_index, root_idxs, emb_table, W_self, W_msg, b):
    src = edge_index[0]
    dst = edge_index[1]
    pad_e = _EPAD - src.shape[0]
    src2d = jnp.concatenate(
        [src, jnp.zeros((pad_e,), jnp.int32)]).reshape(_NW * _CHUNKS, _EC)
    dst2d = jnp.concatenate(
        [dst, jnp.full((pad_e,), N_NODES, jnp.int32)]).reshape(
            _NW * _CHUNKS, _EC)
    wid2d = jnp.concatenate(
        [node_wids, jnp.zeros((_GPAD - N_NODES,), jnp.int32)]).reshape(
            _NW, _GC, _EC)
    zstripe = jnp.zeros((_RPT, HIDDEN), jnp.float32)
    bias = b.reshape(NUM_LAYERS, 1, HIDDEN)

    h0 = _sc_emb_gather(emb_table, wid2d)[:N_NODES]
    hs, hm = _tc_mm_first(h0, W_self[0], W_msg[0], bias[0])
    h_final = None
    for l in range(NUM_LAYERS):
        agg = _sc_edge_agg(hm, src2d, dst2d, zstripe)
        a0 = agg[:N_NODES]
        a1 = agg[_NPAD:_NPAD + N_NODES]
        if l < NUM_LAYERS - 1:
            hs, hm = _tc_mm_mid(hs, a0, a1, W_self[l + 1], W_msg[l + 1],
                                bias[l + 1])
        else:
            h_final = _tc_relu_sum(hs, a0, a1)
    return _sc_root_gather(h_final, root_idxs)


# speedup vs baseline: 7.7088x; 2.6458x over previous
import functools

import jax
import jax.numpy as jnp
from jax import lax
from jax.experimental import pallas as pl
from jax.experimental.pallas import tpu as pltpu
from jax.experimental.pallas import tpu_sc as plsc

N_NODES = 10000
HIDDEN = 128
NUM_LAYERS = 3
BATCH = 256

_NC, _NS = 2, 16
_NW = _NC * _NS
_EC = 128
_CHUNKS = 80
_EPAD = _NW * _CHUNKS * _EC
_NPAD = 10240
_RPT = _NPAD // _NS
_GC = 3
_GPAD = _NW * _GC * _EC
_BPW = BATCH // _NW
_EPW = 10000
_PPW = _CHUNKS * _EC - _EPW

_MESH = plsc.VectorSubcoreMesh(core_axis_name="c", subcore_axis_name="s")


def _f32(*shape):
    return jax.ShapeDtypeStruct(shape, jnp.float32)


def _sc_emb_gather(table, idx2d):
    @functools.partial(
        pl.kernel,
        out_type=_f32(_GPAD, HIDDEN),
        mesh=_MESH,
        scratch_types=[
            pltpu.VMEM((_GC, _EC), jnp.int32),
            pltpu.VMEM((_GC * _EC, HIDDEN), jnp.float32),
            pltpu.SemaphoreType.DMA,
        ],
    )
    def k(tab_hbm, idx_hbm, out_hbm, idx_v, rows_v, sem):
        w = lax.axis_index("s") * _NC + lax.axis_index("c")
        pltpu.sync_copy(idx_hbm.at[w], idx_v)
        for j in range(_GC):
            pltpu.async_copy(tab_hbm.at[idx_v.at[j]],
                             rows_v.at[pl.ds(j * _EC, _EC)], sem)
        for j in range(_GC):
            pltpu.make_async_copy(tab_hbm.at[idx_v.at[j]],
                                  rows_v.at[pl.ds(j * _EC, _EC)], sem).wait()
        pltpu.sync_copy(rows_v, out_hbm.at[pl.ds(w * _GC * _EC, _GC * _EC)])

    return k(table, idx2d)


def _sc_edge_agg(hm, src2d, dst2d, zstripe):
    @functools.partial(
        pl.kernel,
        out_type=_f32(2 * _NPAD, HIDDEN),
        mesh=_MESH,
        scratch_types=[
            pltpu.VMEM((_CHUNKS, _EC), jnp.int32),
            pltpu.VMEM((_CHUNKS, _EC), jnp.int32),
            pltpu.VMEM((_EC, HIDDEN), jnp.float32),
            pltpu.VMEM_SHARED((_NPAD, HIDDEN), jnp.float32),
            pltpu.SemaphoreType.DMA,
        ],
    )
    def k(hm_hbm, src_hbm, dst_hbm, z_hbm, out_hbm, src_v, dst_v, rows_v,
          agg_sh, sem):
        c = lax.axis_index("c")
        s = lax.axis_index("s")
        w = s * _NC + c
        pltpu.sync_copy(z_hbm, agg_sh.at[pl.ds(s * _RPT, _RPT)])
        pltpu.sync_copy(src_hbm.at[pl.ds(w * _CHUNKS, _CHUNKS)], src_v)
        pltpu.sync_copy(dst_hbm.at[pl.ds(w * _CHUNKS, _CHUNKS)], dst_v)
        plsc.subcore_barrier()

        @pl.loop(0, _CHUNKS)
        def _(j):
            pltpu.async_copy(hm_hbm.at[src_v.at[j]], rows_v, sem).wait()
            pltpu.sync_copy(rows_v, agg_sh.at[dst_v.at[j]], add=True)

        plsc.subcore_barrier()
        pltpu.sync_copy(agg_sh.at[pl.ds(s * _RPT, _RPT)],
                        out_hbm.at[pl.ds(c * _NPAD + s * _RPT, _RPT)])

    return k(hm, src2d, dst2d, zstripe)


def _sc_root_gather(h, roots):
    @functools.partial(
        pl.kernel,
        out_type=_f32(BATCH, HIDDEN),
        mesh=_MESH,
        scratch_types=[
            pltpu.VMEM((_BPW,), jnp.int32),
            pltpu.VMEM((_BPW, HIDDEN), jnp.float32),
            pltpu.SemaphoreType.DMA,
        ],
    )
    def k(h_hbm, r_hbm, out_hbm, idx_v, rows_v, sem):
        w = lax.axis_index("s") * _NC + lax.axis_index("c")
        pltpu.sync_copy(r_hbm.at[pl.ds(w * _BPW, _BPW)], idx_v)
        pltpu.async_copy(h_hbm.at[idx_v], rows_v, sem).wait()
        pltpu.sync_copy(rows_v, out_hbm.at[pl.ds(w * _BPW, _BPW)])

    return k(h, roots)


_BLK = 1000
_NBLK = N_NODES // _BLK

_row_spec = pl.BlockSpec((_BLK, HIDDEN), lambda i: (i, 0))
_w_spec = pl.BlockSpec((HIDDEN, HIDDEN), lambda i: (0, 0))
_b_spec = pl.BlockSpec((1, HIDDEN), lambda i: (0, 0))


def _tc_mm_first(h0, Ws, Wm, bias):
    def body(h_ref, ws_ref, wm_ref, b_ref, hs_ref, hm_ref):
        hb = h_ref[...]
        hs_ref[...] = (jnp.dot(hb, ws_ref[...],
                               preferred_element_type=jnp.float32)
                       + b_ref[...])
        hm_ref[...] = jnp.dot(hb, wm_ref[...],
                              preferred_element_type=jnp.float32)

    return pl.pallas_call(
        body,
        grid=(_NBLK,),
        in_specs=[_row_spec, _w_spec, _w_spec, _b_spec],
        out_specs=[_row_spec, _row_spec],
        out_shape=[_f32(N_NODES, HIDDEN)] * 2,
    )(h0, Ws, Wm, bias)


def _tc_mm_mid(hs_in, a0, a1, Ws, Wm, bias):
    def body(hs_ref, a0_ref, a1_ref, ws_ref, wm_ref, b_ref, hs_o, hm_o):
        h = jnp.maximum(hs_ref[...] + a0_ref[...] + a1_ref[...], 0.0)
        hs_o[...] = (jnp.dot(h, ws_ref[...],
                             preferred_element_type=jnp.float32)
                     + b_ref[...])
        hm_o[...] = jnp.dot(h, wm_ref[...],
                            preferred_element_type=jnp.float32)

    return pl.pallas_call(
        body,
        grid=(_NBLK,),
        in_specs=[_row_spec, _row_spec, _row_spec, _w_spec, _w_spec, _b_spec],
        out_specs=[_row_spec, _row_spec],
        out_shape=[_f32(N_NODES, HIDDEN)] * 2,
    )(hs_in, a0, a1, Ws, Wm, bias)


def _tc_relu_sum(hs_in, a0, a1):
    def body(hs_ref, a0_ref, a1_ref, h_o):
        h_o[...] = jnp.maximum(hs_ref[...] + a0_ref[...] + a1_ref[...], 0.0)

    return pl.pallas_call(
        body,
        grid=(_NBLK,),
        in_specs=[_row_spec, _row_spec, _row_spec],
        out_specs=_row_spec,
        out_shape=_f32(N_NODES, HIDDEN),
    )(hs_in, a0, a1)


def kernel(node_wids, edge_index, root_idxs, emb_table, W_self, W_msg, b):
    src = edge_index[0]
    dst = edge_index[1]
    pad_src = jnp.broadcast_to(jnp.arange(_PPW, dtype=jnp.int32)[None, :],
                               (_NW, _PPW))
    pad_dst = jnp.broadcast_to(
        (N_NODES + jnp.arange(_PPW, dtype=jnp.int32))[None, :], (_NW, _PPW))
    src2d = jnp.concatenate([src.reshape(_NW, _EPW), pad_src],
                            axis=1).reshape(_NW * _CHUNKS, _EC)
    dst2d = jnp.concatenate([dst.reshape(_NW, _EPW), pad_dst],
                            axis=1).reshape(_NW * _CHUNKS, _EC)
    wid_pad = jnp.arange(_GPAD - N_NODES, dtype=jnp.int32) % 779
    wid2d = jnp.concatenate([node_wids, wid_pad]).reshape(_NW, _GC, _EC)
    zstripe = jnp.zeros((_RPT, HIDDEN), jnp.float32)
    bias = b.reshape(NUM_LAYERS, 1, HIDDEN)

    h0 = _sc_emb_gather(emb_table, wid2d)[:N_NODES]
    hs, hm = _tc_mm_first(h0, W_self[0], W_msg[0], bias[0])
    h_final = None
    for l in range(NUM_LAYERS):
        agg = _sc_edge_agg(hm, src2d, dst2d, zstripe)
        a0 = agg[:N_NODES]
        a1 = agg[_NPAD:_NPAD + N_NODES]
        if l < NUM_LAYERS - 1:
            hs, hm = _tc_mm_mid(hs, a0, a1, W_self[l + 1], W_msg[l + 1],
                                bias[l + 1])
        else:
            h_final = _tc_relu_sum(hs, a0, a1)
    return _sc_root_gather(h_final, root_idxs)
